# baseline (speedup 1.0000x reference)
import functools

import jax
import jax.numpy as jnp
from jax import lax
from jax.experimental import pallas as pl
from jax.experimental.pallas import tpu as pltpu
from jax.experimental.pallas import tpu_sc as plsc

B, T, K = 64, 1024, 768
N_MASK, N_UNMASK = 768, 256
M_ROWS = B * N_MASK
U_ROWS = B * N_UNMASK
OUT_ROWS = B * T

NC, NS = 2, 16
NW = NC * NS
CM = 64
CU = 32
M_CHUNKS = M_ROWS // CM
U_CHUNKS = U_ROWS // CU
M_PER_W = M_CHUNKS // NW
U_PER_W = U_CHUNKS // NW
NV = K // 16


def _rsqrt16(v):
    h = v * 0.5
    i = plsc.bitcast(v, jnp.int32)
    i = jnp.int32(0x5F3759DF) - (i >> 1)
    y = plsc.bitcast(i, jnp.float32)
    y = y * (1.5 - h * y * y)
    y = y * (1.5 - h * y * y)
    y = y * (1.5 - h * y * y)
    return y


def _ln_rows(buf, nrows, gamma_v, beta_v):
    def row(r, carry):
        s = jnp.zeros((16,), jnp.float32)
        q = jnp.zeros((16,), jnp.float32)
        for c in range(NV):
            x = buf[r, pl.ds(c * 16, 16)] + buf[r + nrows, pl.ds(c * 16, 16)]
            buf[r, pl.ds(c * 16, 16)] = x
            s = s + x
            q = q + x * x
        mu = jnp.sum(s) * (1.0 / K)
        var = jnp.sum(q) * (1.0 / K) - mu * mu
        rs = _rsqrt16(jnp.full((16,), var + 1e-5, jnp.float32))
        muv = jnp.full((16,), mu, jnp.float32)
        for c in range(NV):
            x = buf[r, pl.ds(c * 16, 16)]
            g = gamma_v[pl.ds(c * 16, 16)]
            b = beta_v[pl.ds(c * 16, 16)]
            buf[r, pl.ds(c * 16, 16)] = (x - muv) * rs * g + b
        return carry

    lax.fori_loop(0, nrows, row, 0)


@functools.partial(
    pl.kernel,
    mesh=plsc.VectorSubcoreMesh(core_axis_name="c", subcore_axis_name="s"),
    out_type=jax.ShapeDtypeStruct((OUT_ROWS, K), jnp.float32),
    scratch_types=[
        pltpu.VMEM((M_PER_W, CM), jnp.int32),
        pltpu.VMEM((U_PER_W, CU), jnp.int32),
        pltpu.VMEM((K,), jnp.float32),
        pltpu.VMEM((K,), jnp.float32),
        pltpu.VMEM((CM, K), jnp.float32),
        pltpu.VMEM((CM, K), jnp.float32),
        pltpu.SemaphoreType.DMA,
        pltpu.SemaphoreType.DMA,
        pltpu.SemaphoreType.DMA,
        pltpu.SemaphoreType.DMA,
    ],
    compiler_params=pltpu.CompilerParams(needs_layout_passes=False),
)
def _sc_assemble(eo_hbm, up_hbm, mask_hbm, gamma_hbm, beta_hbm,
                 midx_hbm, uidx_hbm, out_hbm,
                 midx_v, uidx_v, gamma_v, beta_v, buf0, buf1, l0, l1, s0, s1):
    wid = lax.axis_index("s") * NC + lax.axis_index("c")

    pltpu.sync_copy(midx_hbm.at[pl.ds(wid * M_PER_W, M_PER_W)], midx_v)
    pltpu.sync_copy(uidx_hbm.at[pl.ds(wid * U_PER_W, U_PER_W)], uidx_v)
    pltpu.sync_copy(gamma_hbm, gamma_v)
    pltpu.sync_copy(beta_hbm, beta_v)

    def mload(c, rows, sem):
        pltpu.async_copy(mask_hbm.at[pl.ds(c * CM, CM)], rows, sem)

    def mwait_load(rows, sem):
        pltpu.make_async_copy(mask_hbm.at[pl.ds(0, CM)], rows, sem).wait()

    def mscatter(j, rows, sem):
        pltpu.async_copy(rows, out_hbm.at[midx_v.at[j]], sem)

    def mwait_scatter(j, rows, sem):
        pltpu.make_async_copy(rows, out_hbm.at[midx_v.at[j]], sem).wait()

    mbase = wid * M_PER_W
    mload(mbase, buf0, l0)

    def mpair(p, carry):
        c0 = mbase + 2 * p
        j0 = 2 * p
        mwait_load(buf0, l0)
        mscatter(j0, buf0, s0)

        @pl.when(p > 0)
        def _():
            mwait_scatter(j0 - 1, buf1, s1)

        mload(c0 + 1, buf1, l1)
        mwait_load(buf1, l1)
        mscatter(j0 + 1, buf1, s1)
        mwait_scatter(j0, buf0, s0)

        @pl.when(p + 1 < M_PER_W // 2)
        def _():
            mload(c0 + 2, buf0, l0)

        return carry

    lax.fori_loop(0, M_PER_W // 2, mpair, 0)
    mwait_scatter(M_PER_W - 1, buf1, s1)

    def uload(c, buf, sem):
        pltpu.async_copy(eo_hbm.at[pl.ds(c * CU, CU)], buf.at[pl.ds(0, CU)], sem)
        pltpu.async_copy(up_hbm.at[pl.ds(c * CU, CU)], buf.at[pl.ds(CU, CU)], sem)

    def uwait_load(buf, sem):
        pltpu.make_async_copy(
            eo_hbm.at[pl.ds(0, CU)], buf.at[pl.ds(0, CU)], sem).wait()
        pltpu.make_async_copy(
            up_hbm.at[pl.ds(0, CU)], buf.at[pl.ds(CU, CU)], sem).wait()

    def uscatter(j, buf, sem):
        pltpu.async_copy(buf.at[pl.ds(0, CU)], out_hbm.at[uidx_v.at[j]], sem)

    def uwait_scatter(j, buf, sem):
        pltpu.make_async_copy(
            buf.at[pl.ds(0, CU)], out_hbm.at[uidx_v.at[j]], sem).wait()

    ubase = wid * U_PER_W
    uload(ubase, buf0, l0)

    def upair(p, carry):
        c0 = ubase + 2 * p
        j0 = 2 * p
        uwait_load(buf0, l0)

        @pl.when(p > 0)
        def _():
            uwait_scatter(j0 - 1, buf1, s1)

        uload(c0 + 1, buf1, l1)
        _ln_rows(buf0, CU, gamma_v, beta_v)
        uscatter(j0, buf0, s0)
        uwait_load(buf1, l1)
        _ln_rows(buf1, CU, gamma_v, beta_v)
        uscatter(j0 + 1, buf1, s1)
        uwait_scatter(j0, buf0, s0)

        @pl.when(p + 1 < U_PER_W // 2)
        def _():
            uload(c0 + 2, buf0, l0)

        return carry

    lax.fori_loop(0, U_PER_W // 2, upair, 0)
    uwait_scatter(U_PER_W - 1, buf1, s1)


def kernel(encoder_output, mask_embedding, unmasked_positions, gamma, beta,
           mask_id, unmask_id):
    bofs = (jnp.arange(B, dtype=jnp.int32) * T)[:, None]
    midx = (mask_id.astype(jnp.int32) + bofs).reshape(M_CHUNKS, CM)
    uidx = (unmask_id.astype(jnp.int32) + bofs).reshape(U_CHUNKS, CU)
    dec = _sc_assemble(
        encoder_output.reshape(U_ROWS, K),
        unmasked_positions.reshape(U_ROWS, K),
        mask_embedding.reshape(M_ROWS, K),
        gamma, beta, midx, uidx,
    )
    return dec.reshape(B, T, K)


# speedup vs baseline: 1.7982x; 1.7982x over previous
import functools

import jax
import jax.numpy as jnp
from jax import lax
from jax.experimental import pallas as pl
from jax.experimental.pallas import tpu as pltpu
from jax.experimental.pallas import tpu_sc as plsc

B, T, K = 64, 1024, 768
N_MASK, N_UNMASK = 768, 256
M_ROWS = B * N_MASK
U_ROWS = B * N_UNMASK
OUT_ROWS = B * T

NC, NS = 2, 16
NW = NC * NS
CHUNK = 64
M_CHUNKS = M_ROWS // CHUNK
U_CHUNKS = U_ROWS // CHUNK
M_PER_W = M_CHUNKS // NW
U_PER_W = U_CHUNKS // NW

LN_BLK = 2048


def _ln_body(x_ref, p_ref, g_ref, b_ref, o_ref):
    x = x_ref[...] + p_ref[...]
    mu = jnp.mean(x, axis=-1, keepdims=True)
    xc = x - mu
    var = jnp.mean(xc * xc, axis=-1, keepdims=True)
    o_ref[...] = (xc / jnp.sqrt(var + 1e-5)) * g_ref[...] + b_ref[...]


def _layer_norm_tc(x, pos, gamma, beta):
    return pl.pallas_call(
        _ln_body,
        grid=(U_ROWS // LN_BLK,),
        in_specs=[
            pl.BlockSpec((LN_BLK, K), lambda i: (i, 0)),
            pl.BlockSpec((LN_BLK, K), lambda i: (i, 0)),
            pl.BlockSpec((1, K), lambda i: (0, 0)),
            pl.BlockSpec((1, K), lambda i: (0, 0)),
        ],
        out_specs=pl.BlockSpec((LN_BLK, K), lambda i: (i, 0)),
        out_shape=jax.ShapeDtypeStruct((U_ROWS, K), jnp.float32),
    )(x, pos, gamma.reshape(1, K), beta.reshape(1, K))


def _run_stream(src_hbm, out_hbm, idx_all, rows0, rows1, l0, l1, s0, s1,
                base_chunk, npairs):
    def load(c, rows, sem):
        pltpu.async_copy(src_hbm.at[pl.ds(c * CHUNK, CHUNK)], rows, sem)

    def wait_load(rows, sem):
        pltpu.make_async_copy(src_hbm.at[pl.ds(0, CHUNK)], rows, sem).wait()

    def scatter(j, rows, sem):
        pltpu.async_copy(rows, out_hbm.at[idx_all.at[j]], sem)

    def wait_scatter(j, rows, sem):
        pltpu.make_async_copy(rows, out_hbm.at[idx_all.at[j]], sem).wait()

    load(base_chunk, rows0, l0)

    def pair(p, carry):
        c0 = base_chunk + 2 * p
        j0 = 2 * p
        wait_load(rows0, l0)
        scatter(j0, rows0, s0)

        @pl.when(p > 0)
        def _():
            wait_scatter(j0 - 1, rows1, s1)

        load(c0 + 1, rows1, l1)
        wait_load(rows1, l1)
        scatter(j0 + 1, rows1, s1)
        wait_scatter(j0, rows0, s0)

        @pl.when(p + 1 < npairs)
        def _():
            load(c0 + 2, rows0, l0)

        return carry

    lax.fori_loop(0, npairs, pair, 0)
    wait_scatter(2 * npairs - 1, rows1, s1)


_SC_MESH = plsc.VectorSubcoreMesh(core_axis_name="c", subcore_axis_name="s")


@functools.partial(
    pl.kernel,
    mesh=_SC_MESH,
    out_type=jax.ShapeDtypeStruct((OUT_ROWS, K), jnp.float32),
    scratch_types=[
        pltpu.VMEM((M_PER_W, CHUNK), jnp.int32),
        pltpu.VMEM((CHUNK, K), jnp.float32),
        pltpu.VMEM((CHUNK, K), jnp.float32),
        pltpu.SemaphoreType.DMA,
        pltpu.SemaphoreType.DMA,
        pltpu.SemaphoreType.DMA,
        pltpu.SemaphoreType.DMA,
    ],
)
def _sc_scatter_mask(mask_hbm, midx_hbm, out_hbm,
                     idx_all, rows0, rows1, l0, l1, s0, s1):
    wid = lax.axis_index("s") * NC + lax.axis_index("c")
    pltpu.sync_copy(midx_hbm.at[pl.ds(wid * M_PER_W, M_PER_W)], idx_all)
    _run_stream(mask_hbm, out_hbm, idx_all, rows0, rows1, l0, l1, s0, s1,
                wid * M_PER_W, M_PER_W // 2)


@functools.partial(
    pl.kernel,
    mesh=_SC_MESH,
    out_type=(),
    scratch_types=[
        pltpu.VMEM((U_PER_W, CHUNK), jnp.int32),
        pltpu.VMEM((CHUNK, K), jnp.float32),
        pltpu.VMEM((CHUNK, K), jnp.float32),
        pltpu.SemaphoreType.DMA,
        pltpu.SemaphoreType.DMA,
        pltpu.SemaphoreType.DMA,
        pltpu.SemaphoreType.DMA,
    ],
)
def _sc_scatter_unmask(enc_hbm, uidx_hbm, dec_hbm,
                       idx_all, rows0, rows1, l0, l1, s0, s1):
    wid = lax.axis_index("s") * NC + lax.axis_index("c")
    pltpu.sync_copy(uidx_hbm.at[pl.ds(wid * U_PER_W, U_PER_W)], idx_all)
    _run_stream(enc_hbm, dec_hbm, idx_all, rows0, rows1, l0, l1, s0, s1,
                wid * U_PER_W, U_PER_W // 2)


def kernel(encoder_output, mask_embedding, unmasked_positions, gamma, beta,
           mask_id, unmask_id):
    bofs = (jnp.arange(B, dtype=jnp.int32) * T)[:, None]
    midx = (mask_id.astype(jnp.int32) + bofs).reshape(M_CHUNKS, CHUNK)
    uidx = (unmask_id.astype(jnp.int32) + bofs).reshape(U_CHUNKS, CHUNK)

    dec = _sc_scatter_mask(mask_embedding.reshape(M_ROWS, K), midx)
    enc = _layer_norm_tc(
        encoder_output.reshape(U_ROWS, K),
        unmasked_positions.reshape(U_ROWS, K),
        gamma, beta,
    )
    dec_ref = jax.new_ref(dec)
    _sc_scatter_unmask(enc, uidx, dec_ref)
    return dec_ref[...].reshape(B, T, K)
